# baseline (speedup 1.0000x reference)
import functools

import jax
import jax.numpy as jnp
from jax import lax
from jax.experimental import pallas as pl
from jax.experimental.pallas import tpu as pltpu
from jax.experimental.pallas import tpu_sc as plsc

_DIM = 32
_NC = 2
_NS = 16
_NW = _NC * _NS
_CHUNK = 800
_NBUF = 4

_VOCAB = 1000000
_TBLK = 8192


def _transpose_body(wt_ref, w4_ref):
  blk = wt_ref[...]
  t = jnp.transpose(blk)
  w4_ref[...] = t.reshape(_TBLK // 4, 128)


def _transpose_table(wt):
  grid = (_VOCAB + _TBLK - 1) // _TBLK
  return pl.pallas_call(
      _transpose_body,
      grid=(grid,),
      in_specs=[pl.BlockSpec((_DIM, _TBLK), lambda i: (0, i))],
      out_specs=pl.BlockSpec((_TBLK // 4, 128), lambda i: (i, 0)),
      out_shape=jax.ShapeDtypeStruct((_VOCAB // 4, 128), jnp.float32),
  )(wt)


def _make_emb(batch: int):
  b_per_w = batch // _NW
  n_chunks = b_per_w // _CHUNK
  n_groups = n_chunks // _NBUF
  assert n_chunks % _NBUF == 0 and n_groups >= 2
  mesh = plsc.VectorSubcoreMesh(core_axis_name="c", subcore_axis_name="s")

  @functools.partial(
      pl.kernel,
      mesh=mesh,
      out_type=jax.ShapeDtypeStruct((batch, _DIM), jnp.float32),
      scratch_types=[
          pltpu.VMEM((b_per_w,), jnp.int32),
          pltpu.VMEM((_NBUF, _CHUNK, _DIM), jnp.float32),
          pltpu.SemaphoreType.DMA((_NBUF,)),
          pltpu.SemaphoreType.DMA((_NBUF,)),
      ],
      compiler_params=pltpu.CompilerParams(use_tc_tiling_on_sc=False),
  )
  def emb(idx_hbm, table_hbm, out_hbm, idx_full, bufs, gsems, wsems):
    wid = lax.axis_index("s") * _NC + lax.axis_index("c")
    base = wid * b_per_w
    pltpu.sync_copy(idx_hbm.at[pl.ds(base, b_per_w)], idx_full)

    def gather(i, b):
      return pltpu.make_async_copy(
          table_hbm.at[idx_full.at[pl.ds(i * _CHUNK, _CHUNK)]],
          bufs.at[b],
          gsems.at[b],
      )

    def wb(i, b):
      return pltpu.make_async_copy(
          bufs.at[b],
          out_hbm.at[pl.ds(base + i * _CHUNK, _CHUNK)],
          wsems.at[b],
      )

    def slot(i, b, do_a, do_b):
      if do_a:
        wb(i - 1, (b - 1) % _NBUF).wait()
      if do_b:
        gather(i + _NBUF - 1, (b - 1) % _NBUF).start()
      gather(i, b).wait()
      wb(i, b).start()

    for b in range(_NBUF):
      gather(b, b).start()

    slot(0, 0, False, False)
    for b in range(1, _NBUF):
      slot(b, b, True, True)

    def body(g, _):
      i0 = g * _NBUF
      for b in range(_NBUF):
        slot(i0 + b, b, True, True)
      return 0

    lax.fori_loop(1, n_groups - 1, body, 0)

    i0 = (n_groups - 1) * _NBUF
    slot(i0, 0, True, True)
    for b in range(1, _NBUF):
      slot(i0 + b, b, True, False)

    wb(n_chunks - 1, _NBUF - 1).wait()

  return emb


def kernel(x, weight):
  w4 = weight.reshape(_VOCAB // 4, 128)
  w_row = w4.reshape(_VOCAB, _DIM)
  batch = x.size
  out = _make_emb(batch)(x.reshape(-1), w_row)
  return out.reshape(x.shape + (_DIM,))


# speedup vs baseline: 1.0672x; 1.0672x over previous
import functools

import jax
import jax.numpy as jnp
from jax import lax
from jax.experimental import pallas as pl
from jax.experimental.pallas import tpu as pltpu
from jax.experimental.pallas import tpu_sc as plsc

_DIM = 32
_NC = 2
_NS = 16
_NW = _NC * _NS
_CHUNK = 800
_NBUF = 4

_VOCAB = 1000000
_TBLK = 2048
_Q = _TBLK // 4


def _repack_body(wt_ref, w4_ref):
  blk = wt_ref[...]
  t = jnp.transpose(blk)
  w4_ref[...] = jnp.concatenate(
      [t[q * _Q:(q + 1) * _Q, :] for q in range(4)], axis=1)


def _repack_table(wt):
  grid = (_VOCAB + _TBLK - 1) // _TBLK
  return pl.pallas_call(
      _repack_body,
      grid=(grid,),
      in_specs=[pl.BlockSpec((_DIM, _TBLK), lambda i: (0, i))],
      out_specs=pl.BlockSpec((_Q, 128), lambda i: (i, 0)),
      out_shape=jax.ShapeDtypeStruct((_VOCAB // 4, 128), jnp.float32),
  )(wt)


def _make_emb(batch: int):
  b_per_w = batch // _NW
  n_chunks = b_per_w // _CHUNK
  n_groups = n_chunks // _NBUF
  assert n_chunks % _NBUF == 0 and n_groups >= 2
  mesh = plsc.VectorSubcoreMesh(core_axis_name="c", subcore_axis_name="s")

  @functools.partial(
      pl.kernel,
      mesh=mesh,
      out_type=jax.ShapeDtypeStruct((batch, _DIM), jnp.float32),
      scratch_types=[
          pltpu.VMEM((b_per_w,), jnp.int32),
          pltpu.VMEM((_NBUF, _CHUNK, _DIM), jnp.float32),
          pltpu.SemaphoreType.DMA((_NBUF,)),
          pltpu.SemaphoreType.DMA((_NBUF,)),
      ],
      compiler_params=pltpu.CompilerParams(use_tc_tiling_on_sc=False),
  )
  def emb(idx_hbm, table_hbm, out_hbm, idx_full, bufs, gsems, wsems):
    wid = lax.axis_index("s") * _NC + lax.axis_index("c")
    base = wid * b_per_w
    pltpu.sync_copy(idx_hbm.at[pl.ds(base, b_per_w)], idx_full)

    def remap(k, _):
      v = idx_full[pl.ds(k * 16, 16)]
      p = (v & -2048) | ((v & 511) << 2) | ((v >> 9) & 3)
      idx_full[pl.ds(k * 16, 16)] = p
      return 0

    lax.fori_loop(0, b_per_w // 16, remap, 0)

    def gather(i, b):
      return pltpu.make_async_copy(
          table_hbm.at[idx_full.at[pl.ds(i * _CHUNK, _CHUNK)]],
          bufs.at[b],
          gsems.at[b],
      )

    def wb(i, b):
      return pltpu.make_async_copy(
          bufs.at[b],
          out_hbm.at[pl.ds(base + i * _CHUNK, _CHUNK)],
          wsems.at[b],
      )

    def slot(i, b, do_a, do_b):
      if do_a:
        wb(i - 1, (b - 1) % _NBUF).wait()
      if do_b:
        gather(i + _NBUF - 1, (b - 1) % _NBUF).start()
      gather(i, b).wait()
      wb(i, b).start()

    for b in range(_NBUF):
      gather(b, b).start()

    slot(0, 0, False, False)
    for b in range(1, _NBUF):
      slot(b, b, True, True)

    def body(g, _):
      i0 = g * _NBUF
      for b in range(_NBUF):
        slot(i0 + b, b, True, True)
      return 0

    lax.fori_loop(1, n_groups - 1, body, 0)

    i0 = (n_groups - 1) * _NBUF
    slot(i0, 0, True, True)
    for b in range(1, _NBUF):
      slot(i0 + b, b, True, False)

    wb(n_chunks - 1, _NBUF - 1).wait()

  return emb


def kernel(x, weight):
  wt = jnp.transpose(weight)
  w4 = _repack_table(wt)
  w_row = w4.reshape(_VOCAB, _DIM)
  batch = x.size
  out = _make_emb(batch)(x.reshape(-1), w_row)
  return out.reshape(x.shape + (_DIM,))
